# Initial kernel scaffold; baseline (speedup 1.0000x reference)
import functools

import jax
import jax.numpy as jnp
from jax import lax
from jax.experimental import pallas as pl
from jax.experimental.pallas import tpu as pltpu
from jax.experimental.pallas import tpu_sc as plsc

N_NODES = 10000
N_PAD = 10240
N_EDGES = 320000
D_FEAT = 128
HIDDEN = 256
N_CLASSES = 64

NC = 2
NS = 16
NW = NC * NS
EPW = N_EDGES // NW
CHUNK = 80
NCH = EPW // CHUNK
ZR = N_PAD // NS

_GDN = lax.GatherDimensionNumbers(
    offset_dims=(), collapsed_slice_dims=(0,), start_index_map=(0,))


def _bcast_lane(v16, lane):
    idx = jnp.full((16, 1), lane, dtype=jnp.int32)
    return lax.gather(v16, idx, _GDN, (1,),
                      mode=lax.GatherScatterMode.PROMISE_IN_BOUNDS)


def _make_spmm(d_feat, with_deg):
    mesh = plsc.VectorSubcoreMesh(
        core_axis_name="c", subcore_axis_name="s", num_cores=NC,
        num_subcores=NS)
    out_type = [jax.ShapeDtypeStruct((NC, N_PAD, d_feat), jnp.float32)]
    scratch = [
        pltpu.VMEM_SHARED((N_PAD, d_feat), jnp.float32),
        pltpu.VMEM((NCH, CHUNK), jnp.int32),
        pltpu.VMEM((NCH, CHUNK), jnp.int32),
        pltpu.VMEM((NCH, CHUNK), jnp.float32),
        pltpu.VMEM((CHUNK, d_feat), jnp.float32),
        pltpu.SemaphoreType.DMA,
    ]
    if with_deg:
        out_type.append(jax.ShapeDtypeStruct((NC, N_PAD, 16), jnp.float32))
        scratch += [
            pltpu.VMEM_SHARED((N_PAD, 16), jnp.float32),
            pltpu.VMEM((CHUNK, 16), jnp.float32),
        ]

    def body(feat, src3, dst3, w3, zfeat, *rest):
        if with_deg:
            (zdeg, out, deg_out, acc, srcv, dstv, wv, rows, sem,
             dega, wrow) = rest
        else:
            out, acc, srcv, dstv, wv, rows, sem = rest
        cid = lax.axis_index("c")
        sid = lax.axis_index("s")
        wid = sid * NC + cid

        pltpu.sync_copy(zfeat.at[pl.ds(sid * ZR, ZR)],
                        acc.at[pl.ds(sid * ZR, ZR)])
        if with_deg:
            pltpu.sync_copy(zdeg.at[pl.ds(sid * ZR, ZR)],
                            dega.at[pl.ds(sid * ZR, ZR)])
        pltpu.sync_copy(src3.at[wid], srcv)
        pltpu.sync_copy(dst3.at[wid], dstv)
        pltpu.sync_copy(w3.at[wid], wv)
        plsc.subcore_barrier()

        def chunk_body(j, carry):
            pltpu.async_copy(feat.at[srcv.at[j]], rows, sem).wait()

            def grp(g, c2):
                w16 = wv[j, pl.ds(g * 16, 16)]
                for l in range(16):
                    wb = _bcast_lane(w16, l)
                    e = g * 16 + l
                    for k in range(d_feat // 16):
                        rows[e, pl.ds(k * 16, 16)] = (
                            rows[e, pl.ds(k * 16, 16)] * wb)
                    if with_deg:
                        wrow[e, pl.ds(0, 16)] = wb
                return c2

            lax.fori_loop(0, CHUNK // 16, grp, 0)

            pltpu.sync_copy(rows, acc.at[dstv.at[j]], add=True)
            if with_deg:
                pltpu.sync_copy(wrow, dega.at[dstv.at[j]], add=True)
            return carry

        lax.fori_loop(0, NCH, chunk_body, 0)
        plsc.subcore_barrier()

        pltpu.sync_copy(acc.at[pl.ds(sid * ZR, ZR)],
                        out.at[cid, pl.ds(sid * ZR, ZR)])
        if with_deg:
            pltpu.sync_copy(dega.at[pl.ds(sid * ZR, ZR)],
                            deg_out.at[cid, pl.ds(sid * ZR, ZR)])

    return pl.kernel(body, out_type=out_type, mesh=mesh,
                     scratch_types=scratch)


_spmm_deg = _make_spmm(D_FEAT, True)
_spmm_out = _make_spmm(N_CLASSES, False)

_BM = 1024


def _dense_body(p_ref, d_ref, w1_ref, b1_ref, w2_ref, b2_ref, g_ref):
    p = p_ref[0] + p_ref[1]
    d = d_ref[0] + d_ref[1]
    dcol = d[:, 0:1]
    h = jnp.dot(p, w1_ref[...], preferred_element_type=jnp.float32)
    h = jnp.maximum(h + dcol * b1_ref[...], 0.0)
    g = jnp.dot(h, w2_ref[...], preferred_element_type=jnp.float32)
    g_ref[...] = g + b2_ref[...]


_dense = pl.pallas_call(
    _dense_body,
    grid=(N_PAD // _BM,),
    in_specs=[
        pl.BlockSpec((NC, _BM, D_FEAT), lambda i: (0, i, 0)),
        pl.BlockSpec((NC, _BM, 16), lambda i: (0, i, 0)),
        pl.BlockSpec((D_FEAT, HIDDEN), lambda i: (0, 0)),
        pl.BlockSpec((1, HIDDEN), lambda i: (0, 0)),
        pl.BlockSpec((HIDDEN, N_CLASSES), lambda i: (0, 0)),
        pl.BlockSpec((1, N_CLASSES), lambda i: (0, 0)),
    ],
    out_specs=pl.BlockSpec((_BM, N_CLASSES), lambda i: (i, 0)),
    out_shape=jax.ShapeDtypeStruct((N_PAD, N_CLASSES), jnp.float32),
)


def _add_body(q_ref, o_ref):
    o_ref[...] = q_ref[0] + q_ref[1]


_final_add = pl.pallas_call(
    _add_body,
    grid=(N_PAD // _BM,),
    in_specs=[pl.BlockSpec((NC, _BM, N_CLASSES), lambda i: (0, i, 0))],
    out_specs=pl.BlockSpec((_BM, N_CLASSES), lambda i: (i, 0)),
    out_shape=jax.ShapeDtypeStruct((N_PAD, N_CLASSES), jnp.float32),
)


def kernel(X, edge_index, edge_weight, W1, b1, W2, b2):
    idx32 = edge_index.astype(jnp.int32)
    dst3 = idx32[0].reshape(NW, NCH, CHUNK)
    src3 = idx32[1].reshape(NW, NCH, CHUNK)
    w3 = edge_weight.reshape(NW, NCH, CHUNK)
    zfeat = jnp.zeros((N_PAD, D_FEAT), jnp.float32)
    zdeg = jnp.zeros((N_PAD, 16), jnp.float32)
    zout = jnp.zeros((N_PAD, N_CLASSES), jnp.float32)

    p_part, deg_part = _spmm_deg(X, src3, dst3, w3, zfeat, zdeg)
    g = _dense(p_part, deg_part, W1, b1.reshape(1, HIDDEN), W2,
               b2.reshape(1, N_CLASSES))
    q_part = _spmm_out(g, src3, dst3, w3, zout)
    out = _final_add(q_part)
    return out[:N_NODES]


# speedup vs baseline: 6.3383x; 6.3383x over previous
import functools

import jax
import jax.numpy as jnp
from jax import lax
from jax.experimental import pallas as pl
from jax.experimental.pallas import tpu as pltpu
from jax.experimental.pallas import tpu_sc as plsc

N_NODES = 10000
N_PAD = 10240
N_EDGES = 320000
D_FEAT = 128
HIDDEN = 256
N_CLASSES = 64

NC = 2
NS = 16
NW = NC * NS
EPW = N_EDGES // NW
CHUNK = 80
NCH = EPW // CHUNK
ZR = N_PAD // NS

_GDN = lax.GatherDimensionNumbers(
    offset_dims=(), collapsed_slice_dims=(0,), start_index_map=(0,))


def _bcast_lane(v16, lane):
    idx = jnp.full((16, 1), lane, dtype=jnp.int32)
    return lax.gather(v16, idx, _GDN, (1,),
                      mode=lax.GatherScatterMode.PROMISE_IN_BOUNDS)


def _make_spmm(d_feat, with_deg):
    mesh = plsc.VectorSubcoreMesh(
        core_axis_name="c", subcore_axis_name="s", num_cores=NC,
        num_subcores=NS)
    out_type = [jax.ShapeDtypeStruct((NC, N_PAD, d_feat), jnp.float32)]
    scratch = [
        pltpu.VMEM_SHARED((N_PAD, d_feat), jnp.float32),
        pltpu.VMEM((NCH, CHUNK), jnp.int32),
        pltpu.VMEM((NCH, CHUNK), jnp.int32),
        pltpu.VMEM((NCH, CHUNK), jnp.float32),
        pltpu.VMEM((CHUNK, d_feat), jnp.float32),
        pltpu.SemaphoreType.DMA,
    ]
    if with_deg:
        out_type.append(jax.ShapeDtypeStruct((NC, N_PAD), jnp.float32))
        scratch += [
            pltpu.VMEM_SHARED((N_PAD,), jnp.float32),
        ]

    def body(feat, src3, dst3, w3, zfeat, *rest):
        if with_deg:
            (zdeg, out, deg_out, acc, srcv, dstv, wv, rows, sem,
             dega) = rest
        else:
            out, acc, srcv, dstv, wv, rows, sem = rest
        cid = lax.axis_index("c")
        sid = lax.axis_index("s")
        wid = sid * NC + cid

        pltpu.sync_copy(zfeat.at[pl.ds(sid * ZR, ZR)],
                        acc.at[pl.ds(sid * ZR, ZR)])
        if with_deg:
            pltpu.sync_copy(zdeg.at[pl.ds(sid * ZR, ZR)],
                            dega.at[pl.ds(sid * ZR, ZR)])
        pltpu.sync_copy(src3.at[wid], srcv)
        pltpu.sync_copy(dst3.at[wid], dstv)
        pltpu.sync_copy(w3.at[wid], wv)
        plsc.subcore_barrier()

        def chunk_body(j, carry):
            pltpu.async_copy(feat.at[srcv.at[j]], rows, sem).wait()

            def grp(g, c2):
                w16 = wv[j, pl.ds(g * 16, 16)]
                for l in range(16):
                    wb = _bcast_lane(w16, l)
                    e = g * 16 + l
                    for k in range(d_feat // 16):
                        rows[e, pl.ds(k * 16, 16)] = (
                            rows[e, pl.ds(k * 16, 16)] * wb)
                return c2

            lax.fori_loop(0, CHUNK // 16, grp, 0)

            pltpu.sync_copy(rows, acc.at[dstv.at[j]], add=True)
            if with_deg:
                pltpu.sync_copy(wv.at[j], dega.at[dstv.at[j]], add=True)
            return carry

        lax.fori_loop(0, NCH, chunk_body, 0)
        plsc.subcore_barrier()

        pltpu.sync_copy(acc.at[pl.ds(sid * ZR, ZR)],
                        out.at[cid, pl.ds(sid * ZR, ZR)])
        if with_deg:
            pltpu.sync_copy(dega.at[pl.ds(sid * ZR, ZR)],
                            deg_out.at[cid, pl.ds(sid * ZR, ZR)])

    return pl.kernel(body, out_type=out_type, mesh=mesh,
                     scratch_types=scratch,
                     compiler_params=pltpu.CompilerParams(
                         use_tc_tiling_on_sc=False))


_spmm_deg = _make_spmm(D_FEAT, True)
_spmm_out = _make_spmm(N_CLASSES, False)

_BM = 1024


def _dense_body(p_ref, d_ref, w1_ref, b1_ref, w2_ref, b2_ref, g_ref):
    p = p_ref[0] + p_ref[1]
    dcol = d_ref[0] + d_ref[1]
    h = jnp.dot(p, w1_ref[...], preferred_element_type=jnp.float32)
    h = jnp.maximum(h + dcol * b1_ref[...], 0.0)
    g = jnp.dot(h, w2_ref[...], preferred_element_type=jnp.float32)
    g_ref[...] = g + b2_ref[...]


_dense = pl.pallas_call(
    _dense_body,
    grid=(N_PAD // _BM,),
    in_specs=[
        pl.BlockSpec((NC, _BM, D_FEAT), lambda i: (0, i, 0)),
        pl.BlockSpec((NC, _BM, 1), lambda i: (0, i, 0)),
        pl.BlockSpec((D_FEAT, HIDDEN), lambda i: (0, 0)),
        pl.BlockSpec((1, HIDDEN), lambda i: (0, 0)),
        pl.BlockSpec((HIDDEN, N_CLASSES), lambda i: (0, 0)),
        pl.BlockSpec((1, N_CLASSES), lambda i: (0, 0)),
    ],
    out_specs=pl.BlockSpec((_BM, N_CLASSES), lambda i: (i, 0)),
    out_shape=jax.ShapeDtypeStruct((N_PAD, N_CLASSES), jnp.float32),
)


def _add_body(q_ref, o_ref):
    o_ref[...] = q_ref[0] + q_ref[1]


_final_add = pl.pallas_call(
    _add_body,
    grid=(N_PAD // _BM,),
    in_specs=[pl.BlockSpec((NC, _BM, N_CLASSES), lambda i: (0, i, 0))],
    out_specs=pl.BlockSpec((_BM, N_CLASSES), lambda i: (i, 0)),
    out_shape=jax.ShapeDtypeStruct((N_PAD, N_CLASSES), jnp.float32),
)


def kernel(X, edge_index, edge_weight, W1, b1, W2, b2):
    idx32 = edge_index.astype(jnp.int32)
    dst3 = idx32[0].reshape(NW, NCH, CHUNK)
    src3 = idx32[1].reshape(NW, NCH, CHUNK)
    w3 = edge_weight.reshape(NW, NCH, CHUNK)
    zfeat = jnp.zeros((N_PAD, D_FEAT), jnp.float32)
    zdeg = jnp.zeros((N_PAD,), jnp.float32)
    zout = jnp.zeros((N_PAD, N_CLASSES), jnp.float32)

    p_part, deg_part = _spmm_deg(X, src3, dst3, w3, zfeat, zdeg)
    g = _dense(p_part, deg_part.reshape(NC, N_PAD, 1), W1,
               b1.reshape(1, HIDDEN), W2, b2.reshape(1, N_CLASSES))
    [q_part] = _spmm_out(g, src3, dst3, w3, zout)
    out = _final_add(q_part)
    return out[:N_NODES]


# speedup vs baseline: 7.5918x; 1.1978x over previous
import functools

import jax
import jax.numpy as jnp
from jax import lax
from jax.experimental import pallas as pl
from jax.experimental.pallas import tpu as pltpu
from jax.experimental.pallas import tpu_sc as plsc

N_NODES = 10000
N_PAD = 10240
N_EDGES = 320000
D_FEAT = 128
HIDDEN = 256
N_CLASSES = 64

NC = 2
NS = 16
NW = NC * NS
EPW = N_EDGES // NW
CHUNK = 64
EPAD = 48
NCH = (EPW + EPAD) // CHUNK
ZR = N_PAD // NS

_GDN = lax.GatherDimensionNumbers(
    offset_dims=(), collapsed_slice_dims=(0,), start_index_map=(0,))


def _bcast_lane(v16, lane):
    idx = jnp.full((16, 1), lane, dtype=jnp.int32)
    return lax.gather(v16, idx, _GDN, (1,),
                      mode=lax.GatherScatterMode.PROMISE_IN_BOUNDS)


def _make_spmm(d_feat, with_deg):
    mesh = plsc.VectorSubcoreMesh(
        core_axis_name="c", subcore_axis_name="s", num_cores=NC,
        num_subcores=NS)
    out_type = [jax.ShapeDtypeStruct((NC, N_PAD, d_feat), jnp.float32)]
    scratch = [
        pltpu.VMEM_SHARED((N_PAD, d_feat), jnp.float32),
        pltpu.VMEM((NCH, CHUNK), jnp.int32),
        pltpu.VMEM((NCH, CHUNK), jnp.int32),
        pltpu.VMEM((NCH, CHUNK), jnp.float32),
        pltpu.VMEM((CHUNK, d_feat), jnp.float32),
        pltpu.VMEM((CHUNK, d_feat), jnp.float32),
        pltpu.SemaphoreType.DMA,
        pltpu.SemaphoreType.DMA,
        pltpu.SemaphoreType.DMA,
        pltpu.SemaphoreType.DMA,
    ]
    if with_deg:
        out_type.append(jax.ShapeDtypeStruct((NC, N_PAD), jnp.float32))
        scratch += [
            pltpu.VMEM_SHARED((N_PAD,), jnp.float32),
            pltpu.SemaphoreType.DMA,
            pltpu.SemaphoreType.DMA,
        ]

    def body(feat, src3, dst3, w3, zfeat, *rest):
        if with_deg:
            (zdeg, out, deg_out, acc, srcv, dstv, wv, rows0, rows1,
             gsem0, gsem1, ssem0, ssem1, dega, dsem0, dsem1) = rest
            dsem = (dsem0, dsem1)
        else:
            (out, acc, srcv, dstv, wv, rows0, rows1,
             gsem0, gsem1, ssem0, ssem1) = rest
        rows = (rows0, rows1)
        gsem = (gsem0, gsem1)
        ssem = (ssem0, ssem1)
        cid = lax.axis_index("c")
        sid = lax.axis_index("s")
        wid = sid * NC + cid

        z = pltpu.async_copy(zfeat.at[pl.ds(sid * ZR, ZR)],
                             acc.at[pl.ds(sid * ZR, ZR)], gsem0)
        cs = pltpu.async_copy(src3.at[wid], srcv, gsem1)
        cd = pltpu.async_copy(dst3.at[wid], dstv, ssem0)
        cw = pltpu.async_copy(w3.at[wid], wv, ssem1)
        if with_deg:
            zd = pltpu.async_copy(zdeg.at[pl.ds(sid * ZR, ZR)],
                                  dega.at[pl.ds(sid * ZR, ZR)], dsem0)
            zd.wait()
        z.wait()
        cs.wait()
        cd.wait()
        cw.wait()
        plsc.subcore_barrier()

        def fire_g(j, b):
            pltpu.async_copy(feat.at[srcv.at[j]], rows[b], gsem[b])

        def wait_g(j, b):
            pltpu.make_async_copy(feat.at[srcv.at[j]], rows[b],
                                  gsem[b]).wait()

        def fire_s(j, b):
            pltpu.async_copy(rows[b], acc.at[dstv.at[j]], ssem[b], add=True)
            if with_deg:
                pltpu.async_copy(wv.at[j], dega.at[dstv.at[j]], dsem[b],
                                 add=True)

        def wait_s(j, b):
            pltpu.make_async_copy(rows[b], acc.at[dstv.at[j]],
                                  ssem[b]).wait()
            if with_deg:
                pltpu.make_async_copy(wv.at[j], dega.at[dstv.at[j]],
                                      dsem[b]).wait()

        def scale(j, b):
            def grp(g, c2):
                w16 = wv[j, pl.ds(g * 16, 16)]
                for l in range(16):
                    wb = _bcast_lane(w16, l)
                    e = g * 16 + l
                    for k in range(d_feat // 16):
                        rows[b][e, pl.ds(k * 16, 16)] = (
                            rows[b][e, pl.ds(k * 16, 16)] * wb)
                return c2

            lax.fori_loop(0, CHUNK // 16, grp, 0)

        def step(j, b, fire_next):
            wait_g(j, b)
            wait_s(j - 1, 1 - b)
            if fire_next:
                fire_g(j + 1, 1 - b)
            scale(j, b)
            fire_s(j, b)

        fire_g(0, 0)
        wait_g(0, 0)
        fire_g(1, 1)
        scale(0, 0)
        fire_s(0, 0)

        def main(jj, c):
            step(2 * jj + 1, 1, True)
            step(2 * jj + 2, 0, True)
            return c

        m_iters = (NCH - 2) // 2
        lax.fori_loop(0, m_iters, main, 0)
        for j in range(2 * m_iters + 1, NCH):
            step(j, j % 2, j < NCH - 1)
        wait_s(NCH - 1, (NCH - 1) % 2)
        plsc.subcore_barrier()

        pltpu.sync_copy(acc.at[pl.ds(sid * ZR, ZR)],
                        out.at[cid, pl.ds(sid * ZR, ZR)])
        if with_deg:
            pltpu.sync_copy(dega.at[pl.ds(sid * ZR, ZR)],
                            deg_out.at[cid, pl.ds(sid * ZR, ZR)])

    return pl.kernel(body, out_type=out_type, mesh=mesh,
                     scratch_types=scratch,
                     compiler_params=pltpu.CompilerParams(
                         use_tc_tiling_on_sc=False))


_spmm_deg = _make_spmm(D_FEAT, True)
_spmm_out = _make_spmm(N_CLASSES, False)

_BM = 1024


def _dense_body(p_ref, d_ref, w1_ref, b1_ref, w2_ref, b2_ref, g_ref):
    p = p_ref[0] + p_ref[1]
    dcol = d_ref[0] + d_ref[1]
    h = jnp.dot(p, w1_ref[...], preferred_element_type=jnp.float32)
    h = jnp.maximum(h + dcol * b1_ref[...], 0.0)
    g = jnp.dot(h, w2_ref[...], preferred_element_type=jnp.float32)
    g_ref[...] = g + b2_ref[...]


_dense = pl.pallas_call(
    _dense_body,
    grid=(N_PAD // _BM,),
    in_specs=[
        pl.BlockSpec((NC, _BM, D_FEAT), lambda i: (0, i, 0)),
        pl.BlockSpec((NC, _BM, 1), lambda i: (0, i, 0)),
        pl.BlockSpec((D_FEAT, HIDDEN), lambda i: (0, 0)),
        pl.BlockSpec((1, HIDDEN), lambda i: (0, 0)),
        pl.BlockSpec((HIDDEN, N_CLASSES), lambda i: (0, 0)),
        pl.BlockSpec((1, N_CLASSES), lambda i: (0, 0)),
    ],
    out_specs=pl.BlockSpec((_BM, N_CLASSES), lambda i: (i, 0)),
    out_shape=jax.ShapeDtypeStruct((N_PAD, N_CLASSES), jnp.float32),
)


def _add_body(q_ref, o_ref):
    o_ref[...] = q_ref[0] + q_ref[1]


_final_add = pl.pallas_call(
    _add_body,
    grid=(N_PAD // _BM,),
    in_specs=[pl.BlockSpec((NC, _BM, N_CLASSES), lambda i: (0, i, 0))],
    out_specs=pl.BlockSpec((_BM, N_CLASSES), lambda i: (i, 0)),
    out_shape=jax.ShapeDtypeStruct((N_PAD, N_CLASSES), jnp.float32),
)


def _prep(x):
    return jnp.pad(x.reshape(NW, EPW),
                   ((0, 0), (0, EPAD))).reshape(NW, NCH, CHUNK)


def kernel(X, edge_index, edge_weight, W1, b1, W2, b2):
    idx32 = edge_index.astype(jnp.int32)
    dst3 = _prep(idx32[0])
    src3 = _prep(idx32[1])
    w3 = _prep(edge_weight)
    zfeat = jnp.zeros((N_PAD, D_FEAT), jnp.float32)
    zdeg = jnp.zeros((N_PAD,), jnp.float32)
    zout = jnp.zeros((N_PAD, N_CLASSES), jnp.float32)

    p_part, deg_part = _spmm_deg(X, src3, dst3, w3, zfeat, zdeg)
    g = _dense(p_part, deg_part.reshape(NC, N_PAD, 1), W1,
               b1.reshape(1, HIDDEN), W2, b2.reshape(1, N_CLASSES))
    [q_part] = _spmm_out(g, src3, dst3, w3, zout)
    out = _final_add(q_part)
    return out[:N_NODES]
